# baseline (speedup 1.0000x reference)
import functools

import jax
import jax.numpy as jnp
from jax import lax
from jax.experimental import pallas as pl
from jax.experimental.pallas import tpu as pltpu
from jax.experimental.pallas import tpu_sc as plsc

_TOPK = 32
_NEG = -3.0e38


def _stage_a(x_ref, xt_ref, w1_ref, b1_ref, w2_ref, b2_ref, qt_ref, xx_ref):
    f32 = jnp.float32
    xb = x_ref[0]
    xtt = xt_ref[0]
    zt = jnp.maximum(
        jax.lax.dot_general(xtt, w1_ref[...], (((1,), (1,)), ((), ())),
                            preferred_element_type=f32) + b1_ref[...], 0.0)
    qt = jax.lax.dot_general(zt, w2_ref[...], (((1,), (1,)), ((), ())),
                             preferred_element_type=f32) + b2_ref[...]
    qt_ref[0] = qt
    xx_ref[0] = jnp.sum(xb * xb, axis=0, keepdims=True)


def _bmax(v, iota16):
    for k in (1, 2, 4, 8):
        v = jnp.maximum(v, v.at[iota16 ^ k].get(mode="promise_in_bounds"))
    return v


def _sc_body(xc_hbm, xx_hbm, qt_hbm, m_hbm,
             xv0, xv1, xv2, xxv, ndv, cmaxv, idxv0, idxv1, qbuf0, qbuf1,
             mbuf, sem0, sem1,
             *, n_points, batch, rows_per_worker):
    f32 = jnp.float32
    nc = 2
    wid = lax.axis_index("s") * nc + lax.axis_index("c")
    wpb = n_points // rows_per_worker
    b = wid // wpb
    r0 = (wid % wpb) * rows_per_worker

    pltpu.sync_copy(xc_hbm.at[3 * b + 0], xv0)
    pltpu.sync_copy(xc_hbm.at[3 * b + 1], xv1)
    pltpu.sync_copy(xc_hbm.at[3 * b + 2], xv2)
    pltpu.sync_copy(xx_hbm.at[b], xxv)

    iota16 = lax.iota(jnp.int32, 16)
    stride16 = iota16 * 16
    n_groups = n_points // 256

    def reduce_qbuf(qbuf, slot):
        rowsplat = jnp.full((16,), 0, jnp.int32) + slot
        for cc8 in range(8):
            acc = jnp.full((16,), _NEG, f32)
            for r in range(_TOPK):
                acc = jnp.maximum(acc, qbuf[r, pl.ds(cc8 * 16, 16)])
            plsc.store_scatter(mbuf, [rowsplat, cc8 * 16 + iota16], acc)

    def row_body(i, _):
        @pl.when(i < rows_per_worker)
        def _():
            n = r0 + i
            nsplat = jnp.full((16,), 0, jnp.int32) + n
            xn0 = plsc.load_gather(xv0, [nsplat])
            xn1 = plsc.load_gather(xv1, [nsplat])
            xn2 = plsc.load_gather(xv2, [nsplat])

            l2reg = jnp.full((16,), _NEG, f32)
            for g in range(n_groups):
                vs = []
                for jj in range(16):
                    base = g * 256 + jj * 16
                    v = xn0 * xv0[pl.ds(base, 16)]
                    v = v + xn1 * xv1[pl.ds(base, 16)]
                    v = v + xn2 * xv2[pl.ds(base, 16)]
                    v = v + v - xxv[pl.ds(base, 16)]
                    ndv[pl.ds(base, 16)] = v
                    vs.append(v)
                while len(vs) > 1:
                    vs = [jnp.maximum(vs[2 * j], vs[2 * j + 1])
                          for j in range(len(vs) // 2)]
                cmaxv[pl.ds(g * 16, 16)] = vs[0]
                l2reg = jnp.where(iota16 == g, _bmax(vs[0], iota16), l2reg)

            def round_body(k, carry):
                l2reg, idx_a, idx_b = carry
                s = _bmax(l2reg, iota16)
                ggv = _bmax(jnp.where(l2reg == s, iota16, -1), iota16)
                cm16 = plsc.load_gather(cmaxv, [ggv * 16 + iota16])
                ccv = _bmax(jnp.where(cm16 == s, iota16, -1), iota16)
                didx = ggv * 256 + stride16 + ccv
                dchunk = plsc.load_gather(ndv, [didx])
                llv = _bmax(jnp.where(dchunk == s, iota16, -1), iota16)
                colv = ggv * 256 + llv * 16 + ccv + b * n_points
                idx_a = jnp.where(iota16 == k, colv, idx_a)
                idx_b = jnp.where(iota16 == k - 16, colv, idx_b)
                dchunk = jnp.where(iota16 == llv, _NEG, dchunk)
                plsc.store_scatter(ndv, [didx], dchunk)
                ncm = _bmax(dchunk, iota16)
                cm16 = jnp.where(iota16 == ccv, ncm, cm16)
                plsc.store_scatter(cmaxv, [ggv * 16 + iota16], cm16)
                nl2 = _bmax(cm16, iota16)
                l2reg = jnp.where(iota16 == ggv, nl2, l2reg)
                return l2reg, idx_a, idx_b

            zi = jnp.zeros((16,), jnp.int32)
            _, idx_a, idx_b = lax.fori_loop(0, _TOPK, round_body,
                                            (l2reg, zi, zi))

            @pl.when(i % 2 == 0)
            def _():
                idxv0[pl.ds(0, 16)] = idx_a
                idxv0[pl.ds(16, 16)] = idx_b
                pltpu.async_copy(qt_hbm.at[idxv0], qbuf0, sem0)

            @pl.when(i % 2 == 1)
            def _():
                idxv1[pl.ds(0, 16)] = idx_a
                idxv1[pl.ds(16, 16)] = idx_b
                pltpu.async_copy(qt_hbm.at[idxv1], qbuf1, sem1)

        @pl.when(i > 0)
        def _():
            slot = (i - 1) % 32

            @pl.when(i % 2 == 1)
            def _():
                pltpu.make_async_copy(qt_hbm.at[idxv0], qbuf0, sem0).wait()
                reduce_qbuf(qbuf0, slot)

            @pl.when(i % 2 == 0)
            def _():
                pltpu.make_async_copy(qt_hbm.at[idxv1], qbuf1, sem1).wait()
                reduce_qbuf(qbuf1, slot)

            @pl.when(slot == 31)
            def _():
                pltpu.sync_copy(
                    mbuf,
                    m_hbm.at[pl.ds(b * n_points + r0 + ((i - 1) // 32) * 32,
                                   32)])
        return 0

    lax.fori_loop(0, rows_per_worker + 1, row_body, 0)


def _stage_c(m_ref, w3_ref, b3_ref, g_ref):
    f32 = jnp.float32
    t = pl.program_id(1)
    mr = jnp.maximum(m_ref[...], 0.0)
    h3 = jnp.maximum(
        jax.lax.dot_general(mr, w3_ref[...], (((1,), (1,)), ((), ())),
                            preferred_element_type=f32) + b3_ref[...], 0.0)
    part = jnp.max(h3, axis=0, keepdims=True)

    @pl.when(t == 0)
    def _():
        g_ref[0] = part

    @pl.when(t > 0)
    def _():
        g_ref[0] = jnp.maximum(g_ref[0], part)


def _stage_d(g_ref, wf1_ref, bf1_ref, wf2_ref, bf2_ref, wf3_ref, bf3_ref,
             x_ref, o_ref, *, c_dim):
    f32 = jnp.float32
    gb = g_ref[0]
    h1 = jnp.maximum(
        jax.lax.dot_general(gb, wf1_ref[...], (((1,), (1,)), ((), ())),
                            preferred_element_type=f32) + bf1_ref[...], 0.0)
    h2 = jnp.maximum(
        jax.lax.dot_general(h1, wf2_ref[...], (((1,), (1,)), ((), ())),
                            preferred_element_type=f32) + bf2_ref[...], 0.0)
    t9 = jax.lax.dot_general(h2, wf3_ref[...], (((1,), (1,)), ((), ())),
                             preferred_element_type=f32) + bf3_ref[...]
    i9 = lax.broadcasted_iota(jnp.int32, (1, c_dim * c_dim), 1)
    t9 = t9 + jnp.where(i9 % (c_dim + 1) == 0, 1.0, 0.0)

    xb = x_ref[0]
    rows = []
    for d in range(c_dim):
        acc = xb[0:1, :] * t9[0:1, d:d + 1]
        for c in range(1, c_dim):
            acc = acc + xb[c:c + 1, :] * t9[0:1, c_dim * c + d:c_dim * c + d + 1]
        rows.append(acc)
    o_ref[0] = jnp.concatenate(rows, axis=0)


def kernel(x, W1, b1, W2, b2, W3, b3, Wf1, bf1, Wf2, bf2, Wf3, bf3):
    B, C, N = x.shape
    f32 = jnp.float32
    xt = jnp.transpose(x, (0, 2, 1))
    xc = x.reshape(B * C, N)
    b1r = b1.reshape(1, -1)
    b2r = b2.reshape(1, -1)
    b3r = b3.reshape(1, -1)

    qt, xx = pl.pallas_call(
        _stage_a,
        grid=(B,),
        in_specs=[
            pl.BlockSpec((1, C, N), lambda b: (b, 0, 0)),
            pl.BlockSpec((1, N, C), lambda b: (b, 0, 0)),
            pl.BlockSpec(W1.shape, lambda b: (0, 0)),
            pl.BlockSpec(b1r.shape, lambda b: (0, 0)),
            pl.BlockSpec(W2.shape, lambda b: (0, 0)),
            pl.BlockSpec(b2r.shape, lambda b: (0, 0)),
        ],
        out_specs=[
            pl.BlockSpec((1, N, 128), lambda b: (b, 0, 0)),
            pl.BlockSpec((1, 1, N), lambda b: (b, 0, 0)),
        ],
        out_shape=[
            jax.ShapeDtypeStruct((B, N, 128), f32),
            jax.ShapeDtypeStruct((B, 1, N), f32),
        ],
    )(x, xt, W1, b1r, W2, b2r)
    qtf = qt.reshape(B * N, 128)
    xx2 = xx.reshape(B, N)

    n_workers = 32
    rpw = (B * N) // n_workers
    mesh = plsc.VectorSubcoreMesh(core_axis_name="c", subcore_axis_name="s")
    sc = functools.partial(
        pl.kernel,
        mesh=mesh,
        compiler_params=pltpu.CompilerParams(needs_layout_passes=False),
        out_type=jax.ShapeDtypeStruct((B * N, 128), f32),
        scratch_types=[
            pltpu.VMEM((N,), f32),
            pltpu.VMEM((N,), f32),
            pltpu.VMEM((N,), f32),
            pltpu.VMEM((N,), f32),
            pltpu.VMEM((N,), f32),
            pltpu.VMEM((N // 16,), f32),
            pltpu.VMEM((_TOPK,), jnp.int32),
            pltpu.VMEM((_TOPK,), jnp.int32),
            pltpu.VMEM((_TOPK, 128), f32),
            pltpu.VMEM((_TOPK, 128), f32),
            pltpu.VMEM((32, 128), f32),
            pltpu.SemaphoreType.DMA,
            pltpu.SemaphoreType.DMA,
        ],
    )(functools.partial(_sc_body, n_points=N, batch=B, rows_per_worker=rpw))
    m = sc(xc, xx2, qtf)

    R2 = 1024
    T2 = (B * N) // R2 // B
    g = pl.pallas_call(
        _stage_c,
        grid=(B, T2),
        in_specs=[
            pl.BlockSpec((R2, 128), lambda b, t: (b * T2 + t, 0)),
            pl.BlockSpec(W3.shape, lambda b, t: (0, 0)),
            pl.BlockSpec(b3r.shape, lambda b, t: (0, 0)),
        ],
        out_specs=pl.BlockSpec((1, 1, 1024), lambda b, t: (b, 0, 0)),
        out_shape=jax.ShapeDtypeStruct((B, 1, 1024), f32),
    )(m, W3, b3r)

    bf1r = bf1.reshape(1, -1)
    bf2r = bf2.reshape(1, -1)
    bf3r = bf3.reshape(1, -1)
    out = pl.pallas_call(
        functools.partial(_stage_d, c_dim=C),
        grid=(B,),
        in_specs=[
            pl.BlockSpec((1, 1, 1024), lambda b: (b, 0, 0)),
            pl.BlockSpec(Wf1.shape, lambda b: (0, 0)),
            pl.BlockSpec(bf1r.shape, lambda b: (0, 0)),
            pl.BlockSpec(Wf2.shape, lambda b: (0, 0)),
            pl.BlockSpec(bf2r.shape, lambda b: (0, 0)),
            pl.BlockSpec(Wf3.shape, lambda b: (0, 0)),
            pl.BlockSpec(bf3r.shape, lambda b: (0, 0)),
            pl.BlockSpec((1, C, N), lambda b: (b, 0, 0)),
        ],
        out_specs=pl.BlockSpec((1, C, N), lambda b: (b, 0, 0)),
        out_shape=jax.ShapeDtypeStruct((B, C, N), f32),
    )(g, Wf1, bf1r, Wf2, bf2r, Wf3, bf3r, x)
    return out


# speedup vs baseline: 2.6487x; 2.6487x over previous
import functools

import jax
import jax.numpy as jnp
from jax import lax
from jax.experimental import pallas as pl
from jax.experimental.pallas import tpu as pltpu
from jax.experimental.pallas import tpu_sc as plsc

_TOPK = 32
_NEG = -3.0e38


def _stage_a(x_ref, xt_ref, w1_ref, b1_ref, w2_ref, b2_ref, qt_ref, xx_ref):
    f32 = jnp.float32
    xb = x_ref[0]
    xtt = xt_ref[0]
    zt = jnp.maximum(
        jax.lax.dot_general(xtt, w1_ref[...], (((1,), (1,)), ((), ())),
                            preferred_element_type=f32) + b1_ref[...], 0.0)
    qt = jax.lax.dot_general(zt, w2_ref[...], (((1,), (1,)), ((), ())),
                             preferred_element_type=f32) + b2_ref[...]
    qt_ref[0] = qt
    xx_ref[0] = jnp.sum(xb * xb, axis=0, keepdims=True)


def _bmax(v, iota16):
    for k in (1, 2, 4, 8):
        v = jnp.maximum(v, v.at[iota16 ^ k].get(mode="promise_in_bounds"))
    return v


def _sc_body(xc_hbm, xx_hbm, qt_hbm, m_hbm,
             xv0, xv1, xv2, xxv, ndvA, ndvB, ndvC, ndvD,
             cmA, cmB, cmC, cmD,
             idq0, idq1, qq0, qq1,
             mbuf, sq0, sq1,
             *, n_points, batch, rows_per_worker):
    f32 = jnp.float32
    nc = 2
    wid = lax.axis_index("s") * nc + lax.axis_index("c")
    wpb = n_points // rows_per_worker
    b = wid // wpb
    r0 = (wid % wpb) * rows_per_worker

    pltpu.sync_copy(xc_hbm.at[3 * b + 0], xv0)
    pltpu.sync_copy(xc_hbm.at[3 * b + 1], xv1)
    pltpu.sync_copy(xc_hbm.at[3 * b + 2], xv2)
    pltpu.sync_copy(xx_hbm.at[b], xxv)

    iota16 = lax.iota(jnp.int32, 16)
    stride16 = iota16 * 16
    n_groups = n_points // 256
    n_pairs = rows_per_worker // 2

    def reduce_qbuf(qbuf, rowbase, slot):
        rowsplat = jnp.full((16,), 0, jnp.int32) + slot
        for cc8 in range(8):
            acc = jnp.full((16,), _NEG, f32)
            for r in range(_TOPK):
                acc = jnp.maximum(acc, qbuf[rowbase + r, pl.ds(cc8 * 16, 16)])
            plsc.store_scatter(mbuf, [rowsplat, cc8 * 16 + iota16], acc)

    def splat3(n):
        nsplat = jnp.full((16,), 0, jnp.int32) + n
        return (plsc.load_gather(xv0, [nsplat]),
                plsc.load_gather(xv1, [nsplat]),
                plsc.load_gather(xv2, [nsplat]))

    fiota = iota16.astype(jnp.float32)

    def one_round(k, st, ndv, cmv, idx_a, idx_b):
        l2reg = st
        s = _bmax(l2reg, iota16)
        ggf = _bmax(jnp.where(l2reg == s, fiota, -1.0), iota16)
        cmidx = (ggf * 16.0).astype(jnp.int32) + iota16
        cm16 = plsc.load_gather(cmv, [cmidx])
        ccf = _bmax(jnp.where(cm16 == s, fiota, -1.0), iota16)
        didx = (ggf * 256.0 + ccf).astype(jnp.int32) + stride16
        dchunk = plsc.load_gather(ndv, [didx])
        llf = _bmax(jnp.where(dchunk == s, fiota, -1.0), iota16)
        colv = ggf * 256.0 + llf * 16.0 + ccf
        idx_a = jnp.where(iota16 == k, colv, idx_a)
        idx_b = jnp.where(iota16 == k - 16, colv, idx_b)
        dchunk = jnp.where(fiota == llf, _NEG, dchunk)
        plsc.store_scatter(ndv, [didx], dchunk)
        ncm = _bmax(dchunk, iota16)
        cm16 = jnp.where(fiota == ccf, ncm, cm16)
        plsc.store_scatter(cmv, [cmidx], cm16)
        nl2 = _bmax(cm16, iota16)
        l2reg = jnp.where(fiota == ggf, nl2, l2reg)
        return l2reg, idx_a, idx_b

    n_quads = rows_per_worker // 4

    def quad_body(j, _):
        @pl.when(j < n_quads)
        def _():
            na = r0 + 4 * j
            sp = [splat3(na + r) for r in range(4)]
            sp = [(s0 + s0, s1 + s1, s2 + s2) for (s0, s1, s2) in sp]

            l2 = [jnp.full((16,), _NEG, f32) for _ in range(4)]
            nds = [ndvA, ndvB, ndvC, ndvD]
            cms = [cmA, cmB, cmC, cmD]

            def group_body(g, l2c):
                gb = g * 256
                cmacc = [jnp.full((16,), _NEG, f32) for _ in range(4)]
                for jj in range(16):
                    base = gb + jj * 16
                    c0 = xv0[pl.ds(base, 16)]
                    c1 = xv1[pl.ds(base, 16)]
                    c2 = xv2[pl.ds(base, 16)]
                    cx = xxv[pl.ds(base, 16)]
                    for r in range(4):
                        v = sp[r][0] * c0 + sp[r][1] * c1 + sp[r][2] * c2 - cx
                        nds[r][pl.ds(base, 16)] = v
                        cmacc[r] = jnp.maximum(cmacc[r], v)
                out = []
                for r in range(4):
                    cms[r][pl.ds(g * 16, 16)] = cmacc[r]
                    out.append(jnp.where(iota16 == g,
                                         _bmax(cmacc[r], iota16), l2c[r]))
                return tuple(out)

            l2 = lax.fori_loop(0, n_groups, group_body, tuple(l2))

            def round_body(k, carry):
                st = list(carry)
                for r in range(4):
                    a, ia, ib = one_round(k, st[3 * r], nds[r], cms[r],
                                          st[3 * r + 1], st[3 * r + 2])
                    st[3 * r], st[3 * r + 1], st[3 * r + 2] = a, ia, ib
                return tuple(st)

            zf = jnp.zeros((16,), jnp.float32)
            init = []
            for r in range(4):
                init += [l2[r], zf, zf]
            fin = lax.fori_loop(0, _TOPK, round_body, tuple(init))
            bofs = b * n_points

            @pl.when(j % 2 == 0)
            def _():
                for r in range(4):
                    idq0[pl.ds(r * 32, 16)] = (
                        fin[3 * r + 1].astype(jnp.int32) + bofs)
                    idq0[pl.ds(r * 32 + 16, 16)] = (
                        fin[3 * r + 2].astype(jnp.int32) + bofs)
                pltpu.async_copy(qt_hbm.at[idq0], qq0, sq0)

            @pl.when(j % 2 == 1)
            def _():
                for r in range(4):
                    idq1[pl.ds(r * 32, 16)] = (
                        fin[3 * r + 1].astype(jnp.int32) + bofs)
                    idq1[pl.ds(r * 32 + 16, 16)] = (
                        fin[3 * r + 2].astype(jnp.int32) + bofs)
                pltpu.async_copy(qt_hbm.at[idq1], qq1, sq1)

        @pl.when(j > 0)
        def _():
            sl = (4 * j - 4) % 32

            @pl.when(j % 2 == 1)
            def _():
                pltpu.make_async_copy(qt_hbm.at[idq0], qq0, sq0).wait()
                for r in range(4):
                    reduce_qbuf(qq0, r * 32, sl + r)

            @pl.when(j % 2 == 0)
            def _():
                pltpu.make_async_copy(qt_hbm.at[idq1], qq1, sq1).wait()
                for r in range(4):
                    reduce_qbuf(qq1, r * 32, sl + r)

            @pl.when(sl == 28)
            def _():
                pltpu.sync_copy(
                    mbuf,
                    m_hbm.at[pl.ds(b * n_points + r0 + ((4 * j - 4) // 32) * 32,
                                   32)])
        return 0

    lax.fori_loop(0, n_quads + 1, quad_body, 0)


def _stage_c(m_ref, w3_ref, b3_ref, g_ref):
    f32 = jnp.float32
    t = pl.program_id(1)
    mr = jnp.maximum(m_ref[...], 0.0)
    h3 = jnp.maximum(
        jax.lax.dot_general(mr, w3_ref[...], (((1,), (1,)), ((), ())),
                            preferred_element_type=f32) + b3_ref[...], 0.0)
    part = jnp.max(h3, axis=0, keepdims=True)

    @pl.when(t == 0)
    def _():
        g_ref[0] = part

    @pl.when(t > 0)
    def _():
        g_ref[0] = jnp.maximum(g_ref[0], part)


def _stage_d(g_ref, wf1_ref, bf1_ref, wf2_ref, bf2_ref, wf3_ref, bf3_ref,
             x_ref, o_ref, *, c_dim):
    f32 = jnp.float32
    gb = g_ref[0]
    h1 = jnp.maximum(
        jax.lax.dot_general(gb, wf1_ref[...], (((1,), (1,)), ((), ())),
                            preferred_element_type=f32) + bf1_ref[...], 0.0)
    h2 = jnp.maximum(
        jax.lax.dot_general(h1, wf2_ref[...], (((1,), (1,)), ((), ())),
                            preferred_element_type=f32) + bf2_ref[...], 0.0)
    t9 = jax.lax.dot_general(h2, wf3_ref[...], (((1,), (1,)), ((), ())),
                             preferred_element_type=f32) + bf3_ref[...]
    i9 = lax.broadcasted_iota(jnp.int32, (1, c_dim * c_dim), 1)
    t9 = t9 + jnp.where(i9 % (c_dim + 1) == 0, 1.0, 0.0)

    xb = x_ref[0]
    rows = []
    for d in range(c_dim):
        acc = xb[0:1, :] * t9[0:1, d:d + 1]
        for c in range(1, c_dim):
            acc = acc + xb[c:c + 1, :] * t9[0:1, c_dim * c + d:c_dim * c + d + 1]
        rows.append(acc)
    o_ref[0] = jnp.concatenate(rows, axis=0)


def kernel(x, W1, b1, W2, b2, W3, b3, Wf1, bf1, Wf2, bf2, Wf3, bf3):
    B, C, N = x.shape
    f32 = jnp.float32
    xt = jnp.transpose(x, (0, 2, 1))
    xc = x.reshape(B * C, N)
    b1r = b1.reshape(1, -1)
    b2r = b2.reshape(1, -1)
    b3r = b3.reshape(1, -1)

    qt, xx = pl.pallas_call(
        _stage_a,
        grid=(B,),
        in_specs=[
            pl.BlockSpec((1, C, N), lambda b: (b, 0, 0)),
            pl.BlockSpec((1, N, C), lambda b: (b, 0, 0)),
            pl.BlockSpec(W1.shape, lambda b: (0, 0)),
            pl.BlockSpec(b1r.shape, lambda b: (0, 0)),
            pl.BlockSpec(W2.shape, lambda b: (0, 0)),
            pl.BlockSpec(b2r.shape, lambda b: (0, 0)),
        ],
        out_specs=[
            pl.BlockSpec((1, N, 128), lambda b: (b, 0, 0)),
            pl.BlockSpec((1, 1, N), lambda b: (b, 0, 0)),
        ],
        out_shape=[
            jax.ShapeDtypeStruct((B, N, 128), f32),
            jax.ShapeDtypeStruct((B, 1, N), f32),
        ],
    )(x, xt, W1, b1r, W2, b2r)
    qtf = qt.reshape(B * N, 128)
    xx2 = xx.reshape(B, N)

    n_workers = 32
    rpw = (B * N) // n_workers
    mesh = plsc.VectorSubcoreMesh(core_axis_name="c", subcore_axis_name="s")
    sc = functools.partial(
        pl.kernel,
        mesh=mesh,
        compiler_params=pltpu.CompilerParams(needs_layout_passes=False),
        out_type=jax.ShapeDtypeStruct((B * N, 128), f32),
        scratch_types=[
            pltpu.VMEM((N,), f32),
            pltpu.VMEM((N,), f32),
            pltpu.VMEM((N,), f32),
            pltpu.VMEM((N,), f32),
            pltpu.VMEM((N,), f32),
            pltpu.VMEM((N,), f32),
            pltpu.VMEM((N,), f32),
            pltpu.VMEM((N,), f32),
            pltpu.VMEM((N // 16,), f32),
            pltpu.VMEM((N // 16,), f32),
            pltpu.VMEM((N // 16,), f32),
            pltpu.VMEM((N // 16,), f32),
            pltpu.VMEM((4 * _TOPK,), jnp.int32),
            pltpu.VMEM((4 * _TOPK,), jnp.int32),
            pltpu.VMEM((4 * _TOPK, 128), f32),
            pltpu.VMEM((4 * _TOPK, 128), f32),
            pltpu.VMEM((32, 128), f32),
            pltpu.SemaphoreType.DMA,
            pltpu.SemaphoreType.DMA,
        ],
    )(functools.partial(_sc_body, n_points=N, batch=B, rows_per_worker=rpw))
    m = sc(xc, xx2, qtf)

    R2 = 1024
    T2 = (B * N) // R2 // B
    g = pl.pallas_call(
        _stage_c,
        grid=(B, T2),
        in_specs=[
            pl.BlockSpec((R2, 128), lambda b, t: (b * T2 + t, 0)),
            pl.BlockSpec(W3.shape, lambda b, t: (0, 0)),
            pl.BlockSpec(b3r.shape, lambda b, t: (0, 0)),
        ],
        out_specs=pl.BlockSpec((1, 1, 1024), lambda b, t: (b, 0, 0)),
        out_shape=jax.ShapeDtypeStruct((B, 1, 1024), f32),
    )(m, W3, b3r)

    bf1r = bf1.reshape(1, -1)
    bf2r = bf2.reshape(1, -1)
    bf3r = bf3.reshape(1, -1)
    out = pl.pallas_call(
        functools.partial(_stage_d, c_dim=C),
        grid=(B,),
        in_specs=[
            pl.BlockSpec((1, 1, 1024), lambda b: (b, 0, 0)),
            pl.BlockSpec(Wf1.shape, lambda b: (0, 0)),
            pl.BlockSpec(bf1r.shape, lambda b: (0, 0)),
            pl.BlockSpec(Wf2.shape, lambda b: (0, 0)),
            pl.BlockSpec(bf2r.shape, lambda b: (0, 0)),
            pl.BlockSpec(Wf3.shape, lambda b: (0, 0)),
            pl.BlockSpec(bf3r.shape, lambda b: (0, 0)),
            pl.BlockSpec((1, C, N), lambda b: (b, 0, 0)),
        ],
        out_specs=pl.BlockSpec((1, C, N), lambda b: (b, 0, 0)),
        out_shape=jax.ShapeDtypeStruct((B, C, N), f32),
    )(g, Wf1, bf1r, Wf2, bf2r, Wf3, bf3r, x)
    return out
